# Initial kernel scaffold; baseline (speedup 1.0000x reference)
import jax
import jax.numpy as jnp
from jax.experimental import pallas as pl


def kernel(x, k, W_rel0, b_rel0, W_root0, W_rel1, b_rel1, W_root1, W_rel2, b_rel2, W_root2):
    raise NotImplementedError("write your pallas kernel here")


# speedup vs baseline: 13.5835x; 13.5835x over previous
import jax
import jax.numpy as jnp
from jax.experimental import pallas as pl


_KK = 10


def _body(x_ref, xt_ref, Wr0, br0, Wo0, Wr1, br1, Wo1, Wr2, br2, Wo2, out_ref):
    xb = x_ref[0]
    xt = xt_ref[0]
    n, c = xb.shape

    d2 = jnp.zeros((n, n), jnp.float32)
    for ci in range(c):
        df = xb[:, ci:ci + 1] - xt[ci:ci + 1, :]
        d2 = d2 + df * df
    dist = jnp.sqrt(d2)

    rows = jax.lax.broadcasted_iota(jnp.int32, (n, n), 0)
    cols = jax.lax.broadcasted_iota(jnp.int32, (n, n), 1)
    inf = jnp.float32(jnp.inf)
    dist = jnp.where(rows == cols, inf, dist)

    A = jnp.zeros((n, n), jnp.float32)
    for _ in range(_KK):
        m = jnp.min(dist, axis=1, keepdims=True)
        penal = jnp.where(dist == m, cols, n)
        jstar = jnp.min(penal, axis=1, keepdims=True)
        onehot = cols == jstar
        A = A + onehot.astype(jnp.float32)
        dist = jnp.where(onehot, inf, dist)

    f32 = jnp.float32
    h = xb
    for Wr, br, Wo, act in ((Wr0, br0, Wo0, True),
                            (Wr1, br1, Wo1, True),
                            (Wr2, br2, Wo2, False)):
        agg = jax.lax.dot_general(A, h, (((0,), (0,)), ((), ())),
                                  preferred_element_type=f32)
        hn = (jax.lax.dot_general(agg, Wr[...], (((1,), (1,)), ((), ())),
                                  preferred_element_type=f32)
              + jax.lax.dot_general(h, Wo[...], (((1,), (1,)), ((), ())),
                                    preferred_element_type=f32)
              + br[...])
        h = jnp.maximum(hn, 0.0) if act else hn

    out_ref[0] = jax.lax.dot_general(h, h, (((1,), (1,)), ((), ())),
                                     preferred_element_type=f32)


def kernel(x, k, W_rel0, b_rel0, W_root0, W_rel1, b_rel1, W_root1,
           W_rel2, b_rel2, W_root2):
    b, n, c = x.shape
    xt = jnp.swapaxes(x, 1, 2)
    br0 = b_rel0.reshape(1, -1)
    br1 = b_rel1.reshape(1, -1)
    br2 = b_rel2.reshape(1, -1)

    def full(a):
        return pl.BlockSpec(a.shape, lambda i: (0,) * a.ndim)

    out = pl.pallas_call(
        _body,
        grid=(b,),
        in_specs=[
            pl.BlockSpec((1, n, c), lambda i: (i, 0, 0)),
            pl.BlockSpec((1, c, n), lambda i: (i, 0, 0)),
            full(W_rel0), full(br0), full(W_root0),
            full(W_rel1), full(br1), full(W_root1),
            full(W_rel2), full(br2), full(W_root2),
        ],
        out_specs=pl.BlockSpec((1, n, n), lambda i: (i, 0, 0)),
        out_shape=jax.ShapeDtypeStruct((b, n, n), jnp.float32),
    )(x, xt, W_rel0, br0, W_root0, W_rel1, br1, W_root1, W_rel2, br2,
      W_root2)
    return out[:, None, :, :]


# speedup vs baseline: 25.4875x; 1.8764x over previous
import jax
import jax.numpy as jnp
from jax.experimental import pallas as pl


_KK = 10


def _body(x_ref, xt_ref, Wr0, br0, Wo0, Wr1, br1, Wo1, Wr2, br2, Wo2, out_ref):
    xb = x_ref[0]
    xt = xt_ref[0]
    n, c = xb.shape

    G = jax.lax.dot_general(xb, xt, (((1,), (0,)), ((), ())),
                            preferred_element_type=jnp.float32)
    sq = jnp.sum(xb * xb, axis=1, keepdims=True)
    sqT = jnp.sum(xt * xt, axis=0, keepdims=True)
    dist = sq + sqT - (G + G)

    rowsi = jax.lax.broadcasted_iota(jnp.int32, (n, n), 0)
    colsi = jax.lax.broadcasted_iota(jnp.int32, (n, n), 1)
    cols = colsi.astype(jnp.float32)
    inf = jnp.float32(jnp.inf)
    nf = jnp.float32(n)
    dist = jnp.where(rowsi == colsi, inf, dist)

    A = jnp.zeros((n, n), jnp.float32)
    for _ in range(_KK):
        m = jnp.min(dist, axis=1, keepdims=True)
        penal = jnp.where(dist == m, cols, nf)
        jstar = jnp.min(penal, axis=1, keepdims=True)
        onehot = cols == jstar
        A = A + onehot.astype(jnp.float32)
        dist = jnp.where(onehot, inf, dist)

    f32 = jnp.float32
    h = xb
    for Wr, br, Wo, act in ((Wr0, br0, Wo0, True),
                            (Wr1, br1, Wo1, True),
                            (Wr2, br2, Wo2, False)):
        agg = jax.lax.dot_general(A, h, (((0,), (0,)), ((), ())),
                                  preferred_element_type=f32)
        hn = (jax.lax.dot_general(agg, Wr[...], (((1,), (1,)), ((), ())),
                                  preferred_element_type=f32)
              + jax.lax.dot_general(h, Wo[...], (((1,), (1,)), ((), ())),
                                    preferred_element_type=f32)
              + br[...])
        h = jnp.maximum(hn, 0.0) if act else hn

    out_ref[0] = jax.lax.dot_general(h, h, (((1,), (1,)), ((), ())),
                                     preferred_element_type=f32)


def kernel(x, k, W_rel0, b_rel0, W_root0, W_rel1, b_rel1, W_root1,
           W_rel2, b_rel2, W_root2):
    b, n, c = x.shape
    xt = jnp.swapaxes(x, 1, 2)
    br0 = b_rel0.reshape(1, -1)
    br1 = b_rel1.reshape(1, -1)
    br2 = b_rel2.reshape(1, -1)

    def full(a):
        return pl.BlockSpec(a.shape, lambda i: (0,) * a.ndim)

    out = pl.pallas_call(
        _body,
        grid=(b,),
        in_specs=[
            pl.BlockSpec((1, n, c), lambda i: (i, 0, 0)),
            pl.BlockSpec((1, c, n), lambda i: (i, 0, 0)),
            full(W_rel0), full(br0), full(W_root0),
            full(W_rel1), full(br1), full(W_root1),
            full(W_rel2), full(br2), full(W_root2),
        ],
        out_specs=pl.BlockSpec((1, n, n), lambda i: (i, 0, 0)),
        out_shape=jax.ShapeDtypeStruct((b, n, n), jnp.float32),
    )(x, xt, W_rel0, br0, W_root0, W_rel1, br1, W_root1, W_rel2, br2,
      W_root2)
    return out[:, None, :, :]
